# baseline (speedup 1.0000x reference)
import functools

import numpy as np
import jax
import jax.numpy as jnp
from jax import lax
from jax.experimental import pallas as pl
from jax.experimental.pallas import tpu as pltpu
from jax.experimental.pallas import tpu_sc as plsc

_B = 16384
_N_CAT = 26
_N_CONT = 13
_VOCAB1 = 100001
_DIM = 32
_CHUNK = 64
_GROW = 128


def _sc_cat_gather(flat_tab, idxf):
    mesh = plsc.VectorSubcoreMesh(core_axis_name="c", subcore_axis_name="s")
    nc, ns = mesh.num_cores, mesh.num_subcores
    nw = nc * ns
    rw = _B // nw
    nchunk = rw // _CHUNK
    gpc = _CHUNK * _N_CAT // _GROW

    @functools.partial(
        pl.kernel,
        out_type=jax.ShapeDtypeStruct((_B * _N_CAT, _DIM), jnp.float32),
        mesh=mesh,
        scratch_types=[
            pltpu.VMEM((_N_CAT * rw,), jnp.int32),
            pltpu.VMEM((gpc, _GROW), jnp.int32),
            pltpu.VMEM((gpc, _GROW), jnp.int32),
            pltpu.VMEM((_CHUNK * _N_CAT, _DIM), jnp.float32),
            pltpu.VMEM((_CHUNK * _N_CAT, _DIM), jnp.float32),
            pltpu.SemaphoreType.DMA,
            pltpu.SemaphoreType.DMA,
        ],
        compiler_params=pltpu.CompilerParams(use_tc_tiling_on_sc=False),
    )
    def k(tab_hbm, idxf_hbm, out_hbm, idx_v, g0, g1, r0, r1, sem0, sem1):
        wid = lax.axis_index("s") * nc + lax.axis_index("c")
        wbase = wid * rw
        pltpu.sync_copy(idxf_hbm.at[pl.ds(wbase * _N_CAT, rw * _N_CAT)], idx_v)
        iota16 = lax.iota(jnp.int32, 16)
        gb, rb, sems = (g0, g1), (r0, r1), (sem0, sem1)
        gsteps = _CHUNK * _N_CAT // 16

        def build(cb, gbuf):
            def step(t, carry):
                p = t * 16 + iota16
                i = p - ((p * 80660) >> 21) * _N_CAT
                vals = idx_v[pl.ds(cb * (_CHUNK * _N_CAT) + t * 16, 16)]
                g = vals + i * _VOCAB1
                q = t * 16
                gbuf[q >> 7, pl.ds(q & 127, 16)] = g
                return carry

            lax.fori_loop(0, gsteps, step, 0)

        def fire(gbuf, rbuf, sem):
            return [
                pltpu.async_copy(
                    tab_hbm.at[gbuf.at[j]], rbuf.at[pl.ds(j * _GROW, _GROW)], sem
                )
                for j in range(gpc)
            ]

        build(0, gb[0])
        pend = [None, None]
        pend[0] = fire(gb[0], rb[0], sems[0])
        for cb in range(nchunk):
            s = cb % 2
            if cb + 1 < nchunk:
                build(cb + 1, gb[1 - s])
                pend[1 - s] = fire(gb[1 - s], rb[1 - s], sems[1 - s])
            for h in pend[s]:
                h.wait()
            pltpu.sync_copy(
                rb[s],
                out_hbm.at[pl.ds((wbase + cb * _CHUNK) * _N_CAT, _CHUNK * _N_CAT)],
            )

    return k(flat_tab, idxf)


def _cont_body(cont_ref, gam_ref, bet_ref, e_ref, w_ref, b_ref, out_ref):
    x = cont_ref[...]
    mu = jnp.mean(x, axis=1, keepdims=True)
    xc = x - mu
    var = jnp.mean(xc * xc, axis=1, keepdims=True)
    xn = xc * lax.rsqrt(var + 1e-5)
    zg = xn * gam_ref[...] + bet_ref[...]
    ze = jnp.dot(
        zg,
        e_ref[...],
        preferred_element_type=jnp.float32,
        precision=lax.Precision.HIGHEST,
    )
    out_ref[...] = jnp.maximum(ze * w_ref[...] + b_ref[...], 0.0)


_E_CONST = np.repeat(np.eye(_N_CONT, dtype=np.float32), _DIM, axis=1)
_CONT_BLK = 1024
_D_CONT = _N_CONT * _DIM


def _cont_call(cont, gam, bet, e, w_flat, b_flat):
    return pl.pallas_call(
        _cont_body,
        grid=(_B // _CONT_BLK,),
        in_specs=[
            pl.BlockSpec((_CONT_BLK, _N_CONT), lambda i: (i, 0)),
            pl.BlockSpec((1, _N_CONT), lambda i: (0, 0)),
            pl.BlockSpec((1, _N_CONT), lambda i: (0, 0)),
            pl.BlockSpec((_N_CONT, _D_CONT), lambda i: (0, 0)),
            pl.BlockSpec((1, _D_CONT), lambda i: (0, 0)),
            pl.BlockSpec((1, _D_CONT), lambda i: (0, 0)),
        ],
        out_specs=pl.BlockSpec((_CONT_BLK, _D_CONT), lambda i: (i, 0)),
        out_shape=jax.ShapeDtypeStruct((_B, _D_CONT), jnp.float32),
    )(cont, gam, bet, e, w_flat, b_flat)


def kernel(X, emb_tables, cont_weight, cont_bias, ln_gamma, ln_beta):
    idxf = X[:, :_N_CAT].astype(jnp.int32).reshape(_B * _N_CAT)
    flat_tab = emb_tables.reshape(_N_CAT * _VOCAB1, _DIM)
    gathered = _sc_cat_gather(flat_tab, idxf)
    x_cat = gathered.reshape(_B, _N_CAT * _DIM)

    cont = X[:, _N_CAT:]
    x_cont = _cont_call(
        cont,
        ln_gamma.reshape(1, _N_CONT),
        ln_beta.reshape(1, _N_CONT),
        jnp.asarray(_E_CONST),
        cont_weight.reshape(1, _D_CONT),
        cont_bias.reshape(1, _D_CONT),
    )
    return (x_cat, x_cont)


# speedup vs baseline: 28.3436x; 28.3436x over previous
import functools

import numpy as np
import jax
import jax.numpy as jnp
from jax import lax
from jax.experimental import pallas as pl
from jax.experimental.pallas import tpu as pltpu
from jax.experimental.pallas import tpu_sc as plsc

_B = 16384
_N_CAT = 26
_N_CONT = 13
_VOCAB1 = 100001
_DIM = 32
_CCHUNK = 2048
_NCHUNK = _B // _CCHUNK


def _sc_cat_gather(tabt, idxt):
    mesh = plsc.VectorSubcoreMesh(core_axis_name="c", subcore_axis_name="s")
    nc = mesh.num_cores

    @functools.partial(
        pl.kernel,
        out_type=jax.ShapeDtypeStruct((_N_CAT * _DIM, _B), jnp.float32),
        mesh=mesh,
        scratch_types=[
            pltpu.VMEM((_VOCAB1,), jnp.float32),
            pltpu.VMEM((_CCHUNK,), jnp.int32),
            pltpu.VMEM((_CCHUNK,), jnp.float32),
            pltpu.SemaphoreType.DMA,
            pltpu.SemaphoreType.DMA,
        ],
        compiler_params=pltpu.CompilerParams(needs_layout_passes=False),
    )
    def k(tab_hbm, idxt_hbm, out_hbm, trow, ib, ob, sem, osem):
        d = lax.axis_index("s") * nc + lax.axis_index("c")

        for i in range(_N_CAT):
            pltpu.async_copy(tab_hbm.at[i, d], trow, sem).wait()

            def chunk(c, carry):
                pltpu.async_copy(
                    idxt_hbm.at[i, pl.ds(c * _CCHUNK, _CCHUNK)], ib, sem
                ).wait()

                def win(w, carry2):
                    for u in range(8):
                        q = (w * 8 + u) * 16
                        vals = ib[pl.ds(q, 16)]
                        ob[pl.ds(q, 16)] = plsc.load_gather(trow, [vals])
                    return carry2

                lax.fori_loop(0, _CCHUNK // 128, win, 0)
                pltpu.async_copy(
                    ob,
                    out_hbm.at[i * _DIM + d, pl.ds(c * _CCHUNK, _CCHUNK)],
                    osem,
                ).wait()
                return carry

            lax.fori_loop(0, _NCHUNK, chunk, 0)

    return k(tabt, idxt)


def _cont_body(cont_ref, gam_ref, bet_ref, e_ref, w_ref, b_ref, out_ref):
    x = cont_ref[...]
    mu = jnp.mean(x, axis=0, keepdims=True)
    xc = x - mu
    var = jnp.mean(xc * xc, axis=0, keepdims=True)
    xn = xc * lax.rsqrt(var + 1e-5)
    zg = xn * gam_ref[...] + bet_ref[...]
    ze = jnp.dot(
        e_ref[...],
        zg,
        preferred_element_type=jnp.float32,
        precision=lax.Precision.HIGHEST,
    )
    out_ref[...] = jnp.maximum(ze * w_ref[...] + b_ref[...], 0.0)


_E_CONST = np.repeat(np.eye(_N_CONT, dtype=np.float32), _DIM, axis=0)
_CONT_BLK = 2048
_D_CONT = _N_CONT * _DIM


def _cont_call(cont_t, gam, bet, e, w_flat, b_flat):
    return pl.pallas_call(
        _cont_body,
        grid=(_B // _CONT_BLK,),
        in_specs=[
            pl.BlockSpec((_N_CONT, _CONT_BLK), lambda j: (0, j)),
            pl.BlockSpec((_N_CONT, 1), lambda j: (0, 0)),
            pl.BlockSpec((_N_CONT, 1), lambda j: (0, 0)),
            pl.BlockSpec((_D_CONT, _N_CONT), lambda j: (0, 0)),
            pl.BlockSpec((_D_CONT, 1), lambda j: (0, 0)),
            pl.BlockSpec((_D_CONT, 1), lambda j: (0, 0)),
        ],
        out_specs=pl.BlockSpec((_D_CONT, _CONT_BLK), lambda j: (0, j)),
        out_shape=jax.ShapeDtypeStruct((_D_CONT, _B), jnp.float32),
    )(cont_t, gam, bet, e, w_flat, b_flat)


def kernel(X, emb_tables, cont_weight, cont_bias, ln_gamma, ln_beta):
    xt = X.T
    idxt = xt[:_N_CAT].astype(jnp.int32)
    tabt = jnp.transpose(emb_tables, (0, 2, 1))
    x_cat_t = _sc_cat_gather(tabt, idxt)
    x_cat = x_cat_t.T

    cont_t = xt[_N_CAT:]
    x_cont_t = _cont_call(
        cont_t,
        ln_gamma.reshape(_N_CONT, 1),
        ln_beta.reshape(_N_CONT, 1),
        jnp.asarray(_E_CONST),
        cont_weight.reshape(_D_CONT, 1),
        cont_bias.reshape(_D_CONT, 1),
    )
    return (x_cat, x_cont_t.T)


# speedup vs baseline: 44.9043x; 1.5843x over previous
import functools

import numpy as np
import jax
import jax.numpy as jnp
from jax import lax
from jax.experimental import pallas as pl
from jax.experimental.pallas import tpu as pltpu
from jax.experimental.pallas import tpu_sc as plsc

_B = 16384
_N_CAT = 26
_N_CONT = 13
_VOCAB1 = 100001
_DIM = 32
_CCHUNK = 4096
_NCHUNK = _B // _CCHUNK


def _sc_cat_gather(tabt, idxt):
    mesh = plsc.VectorSubcoreMesh(core_axis_name="c", subcore_axis_name="s")
    nc = mesh.num_cores

    @functools.partial(
        pl.kernel,
        out_type=jax.ShapeDtypeStruct((_N_CAT * _DIM, _B), jnp.float32),
        mesh=mesh,
        scratch_types=[
            pltpu.VMEM((_VOCAB1,), jnp.float32),
            pltpu.VMEM((_CCHUNK,), jnp.int32),
            pltpu.VMEM((_CCHUNK,), jnp.int32),
            pltpu.VMEM((_CCHUNK,), jnp.float32),
            pltpu.VMEM((_CCHUNK,), jnp.float32),
            pltpu.SemaphoreType.DMA,
            pltpu.SemaphoreType.DMA,
            pltpu.SemaphoreType.DMA,
        ],
        compiler_params=pltpu.CompilerParams(needs_layout_passes=False),
    )
    def k(tab_hbm, idxt_hbm, out_hbm, trow, i0, i1, o0, o1, rsem, isem, osem):
        d = lax.axis_index("s") * nc + lax.axis_index("c")
        ib, ob = (i0, i1), (o0, o1)
        out_pend = [None, None]
        idx_pend = [None, None]

        def fire_idx(i, c):
            idx_pend[c % 2] = pltpu.async_copy(
                idxt_hbm.at[i, pl.ds(c * _CCHUNK, _CCHUNK)], ib[c % 2], isem
            )

        fire_idx(0, 0)
        for i in range(_N_CAT):
            row_dma = pltpu.async_copy(tab_hbm.at[i, d], trow, rsem)
            row_dma.wait()
            for c in range(_NCHUNK):
                s = c % 2
                idx_pend[s].wait()
                if c + 1 < _NCHUNK:
                    fire_idx(i, c + 1)
                elif i + 1 < _N_CAT:
                    fire_idx(i + 1, 0)
                if out_pend[s] is not None:
                    out_pend[s].wait()

                def win(w, carry2):
                    for u in range(8):
                        q = (w * 8 + u) * 16
                        vals = ib[s][pl.ds(q, 16)]
                        ob[s][pl.ds(q, 16)] = plsc.load_gather(trow, [vals])
                    return carry2

                lax.fori_loop(0, _CCHUNK // 128, win, 0)
                out_pend[s] = pltpu.async_copy(
                    ob[s],
                    out_hbm.at[i * _DIM + d, pl.ds(c * _CCHUNK, _CCHUNK)],
                    osem,
                )
        out_pend[0].wait()
        out_pend[1].wait()

    return k(tabt, idxt)


def _cont_body(cont_ref, gam_ref, bet_ref, e_ref, w_ref, b_ref, out_ref):
    x = cont_ref[...]
    mu = jnp.mean(x, axis=0, keepdims=True)
    xc = x - mu
    var = jnp.mean(xc * xc, axis=0, keepdims=True)
    xn = xc * lax.rsqrt(var + 1e-5)
    zg = xn * gam_ref[...] + bet_ref[...]
    ze = jnp.dot(
        e_ref[...],
        zg,
        preferred_element_type=jnp.float32,
        precision=lax.Precision.HIGHEST,
    )
    out_ref[...] = jnp.maximum(ze * w_ref[...] + b_ref[...], 0.0)


_E_CONST = np.repeat(np.eye(_N_CONT, dtype=np.float32), _DIM, axis=0)
_CONT_BLK = 2048
_D_CONT = _N_CONT * _DIM


def _cont_call(cont_t, gam, bet, e, w_flat, b_flat):
    return pl.pallas_call(
        _cont_body,
        grid=(_B // _CONT_BLK,),
        in_specs=[
            pl.BlockSpec((_N_CONT, _CONT_BLK), lambda j: (0, j)),
            pl.BlockSpec((_N_CONT, 1), lambda j: (0, 0)),
            pl.BlockSpec((_N_CONT, 1), lambda j: (0, 0)),
            pl.BlockSpec((_D_CONT, _N_CONT), lambda j: (0, 0)),
            pl.BlockSpec((_D_CONT, 1), lambda j: (0, 0)),
            pl.BlockSpec((_D_CONT, 1), lambda j: (0, 0)),
        ],
        out_specs=pl.BlockSpec((_D_CONT, _CONT_BLK), lambda j: (0, j)),
        out_shape=jax.ShapeDtypeStruct((_D_CONT, _B), jnp.float32),
    )(cont_t, gam, bet, e, w_flat, b_flat)


def kernel(X, emb_tables, cont_weight, cont_bias, ln_gamma, ln_beta):
    xt = X.T
    idxt = xt[:_N_CAT].astype(jnp.int32)
    tabt = jnp.transpose(emb_tables, (0, 2, 1))
    x_cat_t = _sc_cat_gather(tabt, idxt)
    x_cat = x_cat_t.T

    cont_t = xt[_N_CAT:]
    x_cont_t = _cont_call(
        cont_t,
        ln_gamma.reshape(_N_CONT, 1),
        ln_beta.reshape(_N_CONT, 1),
        jnp.asarray(_E_CONST),
        cont_weight.reshape(_D_CONT, 1),
        cont_bias.reshape(_D_CONT, 1),
    )
    return (x_cat, x_cont_t.T)


# speedup vs baseline: 49.8230x; 1.1095x over previous
import functools

import numpy as np
import jax
import jax.numpy as jnp
from jax import lax
from jax.experimental import pallas as pl
from jax.experimental.pallas import tpu as pltpu
from jax.experimental.pallas import tpu_sc as plsc

_B = 16384
_N_CAT = 26
_N_CONT = 13
_VOCAB1 = 100001
_DIM = 32
_CCHUNK = 4096
_NCHUNK = _B // _CCHUNK


def _sc_cat_gather(tabt, idxt):
    mesh = plsc.VectorSubcoreMesh(core_axis_name="c", subcore_axis_name="s")
    nc = mesh.num_cores

    @functools.partial(
        pl.kernel,
        out_type=jax.ShapeDtypeStruct((_N_CAT * _DIM, _B), jnp.float32),
        mesh=mesh,
        scratch_types=[
            pltpu.VMEM((_VOCAB1,), jnp.float32),
            pltpu.VMEM((_CCHUNK,), jnp.int32),
            pltpu.VMEM((_CCHUNK,), jnp.int32),
            pltpu.VMEM((_CCHUNK,), jnp.float32),
            pltpu.VMEM((_CCHUNK,), jnp.float32),
            pltpu.SemaphoreType.DMA,
            pltpu.SemaphoreType.DMA,
            pltpu.SemaphoreType.DMA,
        ],
        compiler_params=pltpu.CompilerParams(needs_layout_passes=False),
    )
    def k(tab_hbm, idxt_hbm, out_hbm, trow, i0, i1, o0, o1, rsem, isem, osem):
        d = lax.axis_index("s") * nc + lax.axis_index("c")
        ib, ob = (i0, i1), (o0, o1)
        out_pend = [None, None]
        idx_pend = [None, None]

        def fire_idx(i, c):
            idx_pend[c % 2] = pltpu.async_copy(
                idxt_hbm.at[i, pl.ds(c * _CCHUNK, _CCHUNK)], ib[c % 2], isem
            )

        fire_idx(0, 0)
        for i in range(_N_CAT):
            row_dma = pltpu.async_copy(tab_hbm.at[i, d], trow, rsem)
            row_dma.wait()
            for c in range(_NCHUNK):
                s = c % 2
                idx_pend[s].wait()
                if c + 1 < _NCHUNK:
                    fire_idx(i, c + 1)
                elif i + 1 < _N_CAT:
                    fire_idx(i + 1, 0)
                if out_pend[s] is not None:
                    out_pend[s].wait()

                @plsc.parallel_loop(0, _CCHUNK // 16, 1, unroll=8)
                def _(w):
                    q = w * 16
                    vals = ib[s][pl.ds(q, 16)]
                    ob[s][pl.ds(q, 16)] = plsc.load_gather(trow, [vals])
                out_pend[s] = pltpu.async_copy(
                    ob[s],
                    out_hbm.at[i * _DIM + d, pl.ds(c * _CCHUNK, _CCHUNK)],
                    osem,
                )
        out_pend[0].wait()
        out_pend[1].wait()

    return k(tabt, idxt)


def _cont_body(cont_ref, gam_ref, bet_ref, e_ref, w_ref, b_ref, out_ref):
    x = cont_ref[...]
    mu = jnp.mean(x, axis=0, keepdims=True)
    xc = x - mu
    var = jnp.mean(xc * xc, axis=0, keepdims=True)
    xn = xc * lax.rsqrt(var + 1e-5)
    zg = xn * gam_ref[...] + bet_ref[...]
    ze = jnp.dot(
        e_ref[...],
        zg,
        preferred_element_type=jnp.float32,
        precision=lax.Precision.HIGHEST,
    )
    out_ref[...] = jnp.maximum(ze * w_ref[...] + b_ref[...], 0.0)


_E_CONST = np.repeat(np.eye(_N_CONT, dtype=np.float32), _DIM, axis=0)
_CONT_BLK = 2048
_D_CONT = _N_CONT * _DIM


def _cont_call(cont_t, gam, bet, e, w_flat, b_flat):
    return pl.pallas_call(
        _cont_body,
        grid=(_B // _CONT_BLK,),
        in_specs=[
            pl.BlockSpec((_N_CONT, _CONT_BLK), lambda j: (0, j)),
            pl.BlockSpec((_N_CONT, 1), lambda j: (0, 0)),
            pl.BlockSpec((_N_CONT, 1), lambda j: (0, 0)),
            pl.BlockSpec((_D_CONT, _N_CONT), lambda j: (0, 0)),
            pl.BlockSpec((_D_CONT, 1), lambda j: (0, 0)),
            pl.BlockSpec((_D_CONT, 1), lambda j: (0, 0)),
        ],
        out_specs=pl.BlockSpec((_D_CONT, _CONT_BLK), lambda j: (0, j)),
        out_shape=jax.ShapeDtypeStruct((_D_CONT, _B), jnp.float32),
    )(cont_t, gam, bet, e, w_flat, b_flat)


def kernel(X, emb_tables, cont_weight, cont_bias, ln_gamma, ln_beta):
    xt = X.T
    idxt = xt[:_N_CAT].astype(jnp.int32)
    tabt = jnp.transpose(emb_tables, (0, 2, 1))
    x_cat_t = _sc_cat_gather(tabt, idxt)
    x_cat = x_cat_t.T

    cont_t = xt[_N_CAT:]
    x_cont_t = _cont_call(
        cont_t,
        ln_gamma.reshape(_N_CONT, 1),
        ln_beta.reshape(_N_CONT, 1),
        jnp.asarray(_E_CONST),
        cont_weight.reshape(_D_CONT, 1),
        cont_bias.reshape(_D_CONT, 1),
    )
    return (x_cat, x_cont_t.T)
